# Initial kernel scaffold; baseline (speedup 1.0000x reference)
import jax
import jax.numpy as jnp
from jax.experimental import pallas as pl


def kernel(features, proposals, image_shapes, W1, b1, W2, b2, Wc, bc, Wb, bb):
    raise NotImplementedError("write your pallas kernel here")


# speedup vs baseline: 1.0067x; 1.0067x over previous
import functools

import jax
import jax.numpy as jnp
from jax.experimental import pallas as pl

N_PROP = 1000
C_FEAT = 256
FH = 50
FW = 50
NUM_CLASSES = 91
POOL = 7
REP = 1024
SCORE_THRESH = 0.05
NMS_THRESH = 0.5
DET_PER_IMG = 100
PRE_NMS_TOPK = 1000
BBOX_W = (10.0, 10.0, 5.0, 5.0)
IMG_H = 800.0
IMG_W = 800.0
SPATIAL_SCALE = FH / IMG_H
BBOX_XFORM_CLIP = 4.135166556742356

M_PAD = 1024
BM = 256
BK = 1792
NK1 = 7
D_IN = C_FEAT * POOL * POOL
HEAD_PAD = 512


def _fc1_kernel(x_ref, w_ref, b_ref, o_ref):
    k = pl.program_id(1)

    @pl.when(k == 0)
    def _():
        o_ref[...] = jnp.zeros_like(o_ref)

    o_ref[...] += jnp.dot(x_ref[...], w_ref[...],
                          preferred_element_type=jnp.float32)

    @pl.when(k == NK1 - 1)
    def _():
        o_ref[...] = jnp.maximum(o_ref[...] + b_ref[...], 0.0)


def _fc2_heads_kernel(x_ref, w2_ref, b2_ref, wh_ref, bh_ref, o_ref):
    x2 = jnp.maximum(
        jnp.dot(x_ref[...], w2_ref[...], preferred_element_type=jnp.float32)
        + b2_ref[...], 0.0)
    o_ref[...] = jnp.dot(x2, wh_ref[...],
                         preferred_element_type=jnp.float32) + bh_ref[...]


@functools.partial(jax.jit, static_argnums=())
def _mlp(x_pad, W1, b1, W2, b2, Wh, bh):
    x1 = pl.pallas_call(
        _fc1_kernel,
        grid=(M_PAD // BM, NK1),
        in_specs=[
            pl.BlockSpec((BM, BK), lambda i, k: (i, k)),
            pl.BlockSpec((BK, REP), lambda i, k: (k, 0)),
            pl.BlockSpec((1, REP), lambda i, k: (0, 0)),
        ],
        out_specs=pl.BlockSpec((BM, REP), lambda i, k: (i, 0)),
        out_shape=jax.ShapeDtypeStruct((M_PAD, REP), jnp.float32),
    )(x_pad, W1, b1.reshape(1, REP))

    heads = pl.pallas_call(
        _fc2_heads_kernel,
        grid=(M_PAD // BM,),
        in_specs=[
            pl.BlockSpec((BM, REP), lambda i: (i, 0)),
            pl.BlockSpec((REP, REP), lambda i: (0, 0)),
            pl.BlockSpec((1, REP), lambda i: (0, 0)),
            pl.BlockSpec((REP, HEAD_PAD), lambda i: (0, 0)),
            pl.BlockSpec((1, HEAD_PAD), lambda i: (0, 0)),
        ],
        out_specs=pl.BlockSpec((BM, HEAD_PAD), lambda i: (i, 0)),
        out_shape=jax.ShapeDtypeStruct((M_PAD, HEAD_PAD), jnp.float32),
    )(x1, W2, b2.reshape(1, REP), Wh, bh.reshape(1, HEAD_PAD))
    return heads


def _bilinear(feat, ys, xs):
    C, H, W = feat.shape
    y0 = jnp.floor(ys)
    x0 = jnp.floor(xs)
    y0i = y0.astype(jnp.int32)
    x0i = x0.astype(jnp.int32)
    wy1 = ys - y0
    wy0 = 1.0 - wy1
    wx1 = xs - x0
    wx0 = 1.0 - wx1
    y0c = jnp.clip(y0i, 0, H - 1)
    y1c = jnp.clip(y0i + 1, 0, H - 1)
    x0c = jnp.clip(x0i, 0, W - 1)
    x1c = jnp.clip(x0i + 1, 0, W - 1)
    v = (feat[:, y0c, x0c] * (wy0 * wx0)[None, :]
         + feat[:, y0c, x1c] * (wy0 * wx1)[None, :]
         + feat[:, y1c, x0c] * (wy1 * wx0)[None, :]
         + feat[:, y1c, x1c] * (wy1 * wx1)[None, :])
    return v


def _roi_align(feat, boxes):
    n = boxes.shape[0]
    x1 = boxes[:, 0] * SPATIAL_SCALE
    y1 = boxes[:, 1] * SPATIAL_SCALE
    x2 = boxes[:, 2] * SPATIAL_SCALE
    y2 = boxes[:, 3] * SPATIAL_SCALE
    bw = jnp.maximum(x2 - x1, 1.0)
    bh = jnp.maximum(y2 - y1, 1.0)
    frac = (jnp.arange(POOL, dtype=jnp.float32) + 0.5) / POOL
    xs = x1[:, None] + bw[:, None] * frac[None, :]
    ys = y1[:, None] + bh[:, None] * frac[None, :]
    ysg = jnp.broadcast_to(ys[:, :, None], (n, POOL, POOL)).reshape(-1)
    xsg = jnp.broadcast_to(xs[:, None, :], (n, POOL, POOL)).reshape(-1)
    vals = _bilinear(feat, ysg, xsg)
    return vals.reshape(feat.shape[0], n, POOL, POOL).transpose(1, 0, 2, 3)


def _decode(rel_codes, props):
    wx, wy, ww, wh = BBOX_W
    widths = props[:, 2] - props[:, 0]
    heights = props[:, 3] - props[:, 1]
    ctr_x = props[:, 0] + 0.5 * widths
    ctr_y = props[:, 1] + 0.5 * heights
    rel = rel_codes.reshape(-1, NUM_CLASSES, 4)
    dx = rel[..., 0] / wx
    dy = rel[..., 1] / wy
    dw = jnp.minimum(rel[..., 2] / ww, BBOX_XFORM_CLIP)
    dh = jnp.minimum(rel[..., 3] / wh, BBOX_XFORM_CLIP)
    pcx = dx * widths[:, None] + ctr_x[:, None]
    pcy = dy * heights[:, None] + ctr_y[:, None]
    pw = jnp.exp(dw) * widths[:, None]
    ph = jnp.exp(dh) * heights[:, None]
    return jnp.stack([pcx - 0.5 * pw, pcy - 0.5 * ph,
                      pcx + 0.5 * pw, pcy + 0.5 * ph], axis=-1)


def _nms_keep(boxes, scores):
    n = boxes.shape[0]
    x1, y1, x2, y2 = boxes[:, 0], boxes[:, 1], boxes[:, 2], boxes[:, 3]
    areas = jnp.maximum(x2 - x1, 0.0) * jnp.maximum(y2 - y1, 0.0)
    keep0 = scores > 0.0
    idx = jnp.arange(n)

    def body(i, keep):
        xx1 = jnp.maximum(x1[i], x1)
        yy1 = jnp.maximum(y1[i], y1)
        xx2 = jnp.minimum(x2[i], x2)
        yy2 = jnp.minimum(y2[i], y2)
        inter = jnp.maximum(xx2 - xx1, 0.0) * jnp.maximum(yy2 - yy1, 0.0)
        iou = inter / (areas[i] + areas - inter + 1e-9)
        suppress = (iou > NMS_THRESH) & (idx > i) & keep[i]
        return keep & jnp.logical_not(suppress)

    return jax.lax.fori_loop(0, n, body, keep0)


def kernel(features, proposals, image_shapes, W1, b1, W2, b2, Wc, bc, Wb, bb):
    feat = features[0]
    roi = _roi_align(feat, proposals)
    x = roi.reshape(roi.shape[0], -1)
    x_pad = jnp.zeros((M_PAD, D_IN), jnp.float32).at[:N_PROP].set(x)

    Wh = jnp.zeros((REP, HEAD_PAD), jnp.float32)
    Wh = Wh.at[:, :NUM_CLASSES].set(Wc)
    Wh = Wh.at[:, NUM_CLASSES:NUM_CLASSES + NUM_CLASSES * 4].set(Wb)
    bh = jnp.zeros((HEAD_PAD,), jnp.float32)
    bh = bh.at[:NUM_CLASSES].set(bc)
    bh = bh.at[NUM_CLASSES:NUM_CLASSES + NUM_CLASSES * 4].set(bb)

    heads = _mlp(x_pad, W1, b1, W2, b2, Wh, bh)
    class_logits = heads[:N_PROP, :NUM_CLASSES]
    box_regression = heads[:N_PROP, NUM_CLASSES:NUM_CLASSES + NUM_CLASSES * 4]

    pred_boxes = _decode(box_regression, proposals)
    scores = jax.nn.softmax(class_logits, axis=-1)
    px1 = jnp.clip(pred_boxes[..., 0], 0.0, IMG_W)
    py1 = jnp.clip(pred_boxes[..., 1], 0.0, IMG_H)
    px2 = jnp.clip(pred_boxes[..., 2], 0.0, IMG_W)
    py2 = jnp.clip(pred_boxes[..., 3], 0.0, IMG_H)
    pred_boxes = jnp.stack([px1, py1, px2, py2], axis=-1)
    boxes = pred_boxes[:, 1:, :].reshape(-1, 4)
    scores_f = scores[:, 1:].reshape(-1)
    labels = jnp.broadcast_to(
        jnp.arange(1, NUM_CLASSES)[None, :],
        (N_PROP, NUM_CLASSES - 1)).reshape(-1)
    ws = boxes[:, 2] - boxes[:, 0]
    hs = boxes[:, 3] - boxes[:, 1]
    valid = (scores_f > SCORE_THRESH) & (ws >= 0.01) & (hs >= 0.01)
    masked = jnp.where(valid, scores_f, -1.0)
    top_s, top_i = jax.lax.top_k(masked, PRE_NMS_TOPK)
    cand_boxes = boxes[top_i]
    cand_labels = labels[top_i]
    cand_scores = top_s
    off = cand_labels.astype(jnp.float32) * (
        jax.lax.stop_gradient(jnp.max(cand_boxes)) + 1.0)
    keep = _nms_keep(cand_boxes + off[:, None], cand_scores)
    final_masked = jnp.where(keep, cand_scores, -1.0)
    fs, fi = jax.lax.top_k(final_masked, DET_PER_IMG)
    return cand_boxes[fi], fs, cand_labels[fi]


# speedup vs baseline: 3.2808x; 3.2589x over previous
import functools

import jax
import jax.numpy as jnp
from jax.experimental import pallas as pl

N_PROP = 1000
C_FEAT = 256
FH = 50
FW = 50
NUM_CLASSES = 91
POOL = 7
REP = 1024
SCORE_THRESH = 0.05
NMS_THRESH = 0.5
DET_PER_IMG = 100
PRE_NMS_TOPK = 1000
BBOX_W = (10.0, 10.0, 5.0, 5.0)
IMG_H = 800.0
IMG_W = 800.0
SPATIAL_SCALE = FH / IMG_H
BBOX_XFORM_CLIP = 4.135166556742356

M_PAD = 1024
BM = 256
BK = 1792
NK1 = 7
D_IN = C_FEAT * POOL * POOL
HEAD_PAD = 512


def _fc1_kernel(x_ref, w_ref, b_ref, o_ref):
    k = pl.program_id(1)

    @pl.when(k == 0)
    def _():
        o_ref[...] = jnp.zeros_like(o_ref)

    o_ref[...] += jnp.dot(x_ref[...], w_ref[...],
                          preferred_element_type=jnp.float32)

    @pl.when(k == NK1 - 1)
    def _():
        o_ref[...] = jnp.maximum(o_ref[...] + b_ref[...], 0.0)


def _fc2_heads_kernel(x_ref, w2_ref, b2_ref, wh_ref, bh_ref, o_ref):
    x2 = jnp.maximum(
        jnp.dot(x_ref[...], w2_ref[...], preferred_element_type=jnp.float32)
        + b2_ref[...], 0.0)
    o_ref[...] = jnp.dot(x2, wh_ref[...],
                         preferred_element_type=jnp.float32) + bh_ref[...]


@functools.partial(jax.jit, static_argnums=())
def _mlp(x_pad, W1, b1, W2, b2, Wh, bh):
    x1 = pl.pallas_call(
        _fc1_kernel,
        grid=(M_PAD // BM, NK1),
        in_specs=[
            pl.BlockSpec((BM, BK), lambda i, k: (i, k)),
            pl.BlockSpec((BK, REP), lambda i, k: (k, 0)),
            pl.BlockSpec((1, REP), lambda i, k: (0, 0)),
        ],
        out_specs=pl.BlockSpec((BM, REP), lambda i, k: (i, 0)),
        out_shape=jax.ShapeDtypeStruct((M_PAD, REP), jnp.float32),
    )(x_pad, W1, b1.reshape(1, REP))

    heads = pl.pallas_call(
        _fc2_heads_kernel,
        grid=(M_PAD // BM,),
        in_specs=[
            pl.BlockSpec((BM, REP), lambda i: (i, 0)),
            pl.BlockSpec((REP, REP), lambda i: (0, 0)),
            pl.BlockSpec((1, REP), lambda i: (0, 0)),
            pl.BlockSpec((REP, HEAD_PAD), lambda i: (0, 0)),
            pl.BlockSpec((1, HEAD_PAD), lambda i: (0, 0)),
        ],
        out_specs=pl.BlockSpec((BM, HEAD_PAD), lambda i: (i, 0)),
        out_shape=jax.ShapeDtypeStruct((M_PAD, HEAD_PAD), jnp.float32),
    )(x1, W2, b2.reshape(1, REP), Wh, bh.reshape(1, HEAD_PAD))
    return heads


def _bilinear(feat, ys, xs):
    C, H, W = feat.shape
    y0 = jnp.floor(ys)
    x0 = jnp.floor(xs)
    y0i = y0.astype(jnp.int32)
    x0i = x0.astype(jnp.int32)
    wy1 = ys - y0
    wy0 = 1.0 - wy1
    wx1 = xs - x0
    wx0 = 1.0 - wx1
    y0c = jnp.clip(y0i, 0, H - 1)
    y1c = jnp.clip(y0i + 1, 0, H - 1)
    x0c = jnp.clip(x0i, 0, W - 1)
    x1c = jnp.clip(x0i + 1, 0, W - 1)
    v = (feat[:, y0c, x0c] * (wy0 * wx0)[None, :]
         + feat[:, y0c, x1c] * (wy0 * wx1)[None, :]
         + feat[:, y1c, x0c] * (wy1 * wx0)[None, :]
         + feat[:, y1c, x1c] * (wy1 * wx1)[None, :])
    return v


def _roi_align(feat, boxes):
    n = boxes.shape[0]
    x1 = boxes[:, 0] * SPATIAL_SCALE
    y1 = boxes[:, 1] * SPATIAL_SCALE
    x2 = boxes[:, 2] * SPATIAL_SCALE
    y2 = boxes[:, 3] * SPATIAL_SCALE
    bw = jnp.maximum(x2 - x1, 1.0)
    bh = jnp.maximum(y2 - y1, 1.0)
    frac = (jnp.arange(POOL, dtype=jnp.float32) + 0.5) / POOL
    xs = x1[:, None] + bw[:, None] * frac[None, :]
    ys = y1[:, None] + bh[:, None] * frac[None, :]
    ysg = jnp.broadcast_to(ys[:, :, None], (n, POOL, POOL)).reshape(-1)
    xsg = jnp.broadcast_to(xs[:, None, :], (n, POOL, POOL)).reshape(-1)
    vals = _bilinear(feat, ysg, xsg)
    return vals.reshape(feat.shape[0], n, POOL, POOL).transpose(1, 0, 2, 3)


def _decode(rel_codes, props):
    wx, wy, ww, wh = BBOX_W
    widths = props[:, 2] - props[:, 0]
    heights = props[:, 3] - props[:, 1]
    ctr_x = props[:, 0] + 0.5 * widths
    ctr_y = props[:, 1] + 0.5 * heights
    rel = rel_codes.reshape(-1, NUM_CLASSES, 4)
    dx = rel[..., 0] / wx
    dy = rel[..., 1] / wy
    dw = jnp.minimum(rel[..., 2] / ww, BBOX_XFORM_CLIP)
    dh = jnp.minimum(rel[..., 3] / wh, BBOX_XFORM_CLIP)
    pcx = dx * widths[:, None] + ctr_x[:, None]
    pcy = dy * heights[:, None] + ctr_y[:, None]
    pw = jnp.exp(dw) * widths[:, None]
    ph = jnp.exp(dh) * heights[:, None]
    return jnp.stack([pcx - 0.5 * pw, pcy - 0.5 * ph,
                      pcx + 0.5 * pw, pcy + 0.5 * ph], axis=-1)


def _nms_loop_kernel(s_ref, keep0_ref, keep_ref):
    iota = jax.lax.broadcasted_iota(jnp.int32, (1, M_PAD), 1)

    def body(i, keep):
        k_i = jnp.sum(jnp.where(iota == i, keep, 0.0))
        row = s_ref[pl.ds(i, 1), :]
        return keep * (1.0 - row * k_i)

    keep_ref[...] = jax.lax.fori_loop(0, N_PROP, body, keep0_ref[...])


def _nms_keep(boxes, scores):
    n = boxes.shape[0]
    x1, y1, x2, y2 = boxes[:, 0], boxes[:, 1], boxes[:, 2], boxes[:, 3]
    areas = jnp.maximum(x2 - x1, 0.0) * jnp.maximum(y2 - y1, 0.0)
    xx1 = jnp.maximum(x1[:, None], x1[None, :])
    yy1 = jnp.maximum(y1[:, None], y1[None, :])
    xx2 = jnp.minimum(x2[:, None], x2[None, :])
    yy2 = jnp.minimum(y2[:, None], y2[None, :])
    inter = jnp.maximum(xx2 - xx1, 0.0) * jnp.maximum(yy2 - yy1, 0.0)
    iou = inter / (areas[:, None] + areas[None, :] - inter + 1e-9)
    idx = jnp.arange(n)
    s = ((iou > NMS_THRESH) & (idx[None, :] > idx[:, None])).astype(jnp.float32)
    s_pad = jnp.zeros((M_PAD, M_PAD), jnp.float32).at[:n, :n].set(s)
    keep0 = jnp.zeros((1, M_PAD), jnp.float32).at[0, :n].set(
        (scores > 0.0).astype(jnp.float32))
    keep = pl.pallas_call(
        _nms_loop_kernel,
        out_shape=jax.ShapeDtypeStruct((1, M_PAD), jnp.float32),
    )(s_pad, keep0)
    return keep[0, :n] > 0.5


def kernel(features, proposals, image_shapes, W1, b1, W2, b2, Wc, bc, Wb, bb):
    feat = features[0]
    roi = _roi_align(feat, proposals)
    x = roi.reshape(roi.shape[0], -1)
    x_pad = jnp.zeros((M_PAD, D_IN), jnp.float32).at[:N_PROP].set(x)

    Wh = jnp.zeros((REP, HEAD_PAD), jnp.float32)
    Wh = Wh.at[:, :NUM_CLASSES].set(Wc)
    Wh = Wh.at[:, NUM_CLASSES:NUM_CLASSES + NUM_CLASSES * 4].set(Wb)
    bh = jnp.zeros((HEAD_PAD,), jnp.float32)
    bh = bh.at[:NUM_CLASSES].set(bc)
    bh = bh.at[NUM_CLASSES:NUM_CLASSES + NUM_CLASSES * 4].set(bb)

    heads = _mlp(x_pad, W1, b1, W2, b2, Wh, bh)
    class_logits = heads[:N_PROP, :NUM_CLASSES]
    box_regression = heads[:N_PROP, NUM_CLASSES:NUM_CLASSES + NUM_CLASSES * 4]

    pred_boxes = _decode(box_regression, proposals)
    scores = jax.nn.softmax(class_logits, axis=-1)
    px1 = jnp.clip(pred_boxes[..., 0], 0.0, IMG_W)
    py1 = jnp.clip(pred_boxes[..., 1], 0.0, IMG_H)
    px2 = jnp.clip(pred_boxes[..., 2], 0.0, IMG_W)
    py2 = jnp.clip(pred_boxes[..., 3], 0.0, IMG_H)
    pred_boxes = jnp.stack([px1, py1, px2, py2], axis=-1)
    boxes = pred_boxes[:, 1:, :].reshape(-1, 4)
    scores_f = scores[:, 1:].reshape(-1)
    labels = jnp.broadcast_to(
        jnp.arange(1, NUM_CLASSES)[None, :],
        (N_PROP, NUM_CLASSES - 1)).reshape(-1)
    ws = boxes[:, 2] - boxes[:, 0]
    hs = boxes[:, 3] - boxes[:, 1]
    valid = (scores_f > SCORE_THRESH) & (ws >= 0.01) & (hs >= 0.01)
    masked = jnp.where(valid, scores_f, -1.0)
    top_s, top_i = jax.lax.top_k(masked, PRE_NMS_TOPK)
    cand_boxes = boxes[top_i]
    cand_labels = labels[top_i]
    cand_scores = top_s
    off = cand_labels.astype(jnp.float32) * (
        jax.lax.stop_gradient(jnp.max(cand_boxes)) + 1.0)
    keep = _nms_keep(cand_boxes + off[:, None], cand_scores)
    final_masked = jnp.where(keep, cand_scores, -1.0)
    fs, fi = jax.lax.top_k(final_masked, DET_PER_IMG)
    return cand_boxes[fi], fs, cand_labels[fi]
